# baseline (speedup 1.0000x reference)
import math

import jax
import jax.numpy as jnp
from jax.experimental import pallas as pl
from jax.experimental.pallas import tpu as pltpu

_SEQ = 2048
_HEADS = 16
_NUM_BUCKETS = 32
_MAX_DISTANCE = 128
_W = 2 * _SEQ
_BLK_I = 128
_TILE = 128


def _bias_body(tbl_ref, out_ref, e_ref):
    h = pl.program_id(0)
    ib = pl.program_id(1)

    @pl.when(ib == 0)
    def _build():
        z = jax.lax.broadcasted_iota(jnp.int32, (1, _W), 1)
        rel = z - _SEQ
        n = -rel
        half = _NUM_BUCKETS // 2
        ret = jnp.where(n < 0, half, 0)
        na = jnp.abs(n)
        max_exact = half // 2
        nsq = (na * na).astype(jnp.float32)
        e = (jax.lax.bitcast_convert_type(nsq, jnp.int32) >> 23) - 127
        val_large = jnp.minimum(max_exact + (e - 6), half - 1)
        bucket = ret + jnp.where(na < max_exact, na, val_large)
        w = jnp.zeros((1, _W), jnp.float32)
        for b in range(_NUM_BUCKETS):
            w = w + jnp.where(bucket == b, tbl_ref[0, 0, b], 0.0)
        wb = jnp.broadcast_to(w, (_BLK_I, _W))
        e_ref[:, :] = pltpu.roll(wb, 0, axis=1, stride=1, stride_axis=0)

    for tj in range(_SEQ // _TILE):
        start = pl.multiple_of(_TILE * (tj + _SEQ // _TILE - ib), _TILE)
        out_ref[0, 0, :, _TILE * tj:_TILE * (tj + 1)] = e_ref[:, pl.ds(start, _TILE)]


def kernel(qk_dots, table):
    del qk_dots
    tbl_t = table.T.reshape(_HEADS, 1, _NUM_BUCKETS)
    out = pl.pallas_call(
        _bias_body,
        grid=(_HEADS, _SEQ // _BLK_I),
        in_specs=[
            pl.BlockSpec((1, 1, _NUM_BUCKETS), lambda h, ib: (h, 0, 0),
                         memory_space=pltpu.SMEM),
        ],
        out_specs=pl.BlockSpec((1, 1, _BLK_I, _SEQ),
                               lambda h, ib: (0, h, ib, 0)),
        out_shape=jax.ShapeDtypeStruct((1, _HEADS, _SEQ, _SEQ), jnp.float32),
        scratch_shapes=[pltpu.VMEM((_BLK_I, _W), jnp.float32)],
    )(tbl_t)
    return out


# speedup vs baseline: 1.6195x; 1.6195x over previous
import math

import jax
import jax.numpy as jnp
from jax.experimental import pallas as pl
from jax.experimental.pallas import tpu as pltpu

_SEQ = 2048
_HEADS = 16
_NUM_BUCKETS = 32
_MAX_DISTANCE = 128
_W = 2 * _SEQ
_BLK_I = 512
_EROWS = 128
_TILE = 128


def _bias_body(tbl_ref, out_ref, e_ref):
    h = pl.program_id(0)
    ib = pl.program_id(1)

    @pl.when(ib == 0)
    def _build():
        z = jax.lax.broadcasted_iota(jnp.int32, (1, _W), 1)
        rel = z - _SEQ
        n = -rel
        half = _NUM_BUCKETS // 2
        ret = jnp.where(n < 0, half, 0)
        na = jnp.abs(n)
        max_exact = half // 2
        nsq = (na * na).astype(jnp.float32)
        e = (jax.lax.bitcast_convert_type(nsq, jnp.int32) >> 23) - 127
        val_large = jnp.minimum(max_exact + (e - 6), half - 1)
        bucket = ret + jnp.where(na < max_exact, na, val_large)
        w = jnp.zeros((1, _W), jnp.float32)
        for b in range(_NUM_BUCKETS):
            w = w + jnp.where(bucket == b, tbl_ref[0, 0, b], 0.0)
        wb = jnp.broadcast_to(w, (_EROWS, _W))
        e_ref[:, :] = pltpu.roll(wb, 0, axis=1, stride=1, stride_axis=0)

    nt = _SEQ // _TILE
    for sub in range(_BLK_I // _TILE):
        gib = (_BLK_I // _TILE) * ib + sub
        r0 = _TILE * sub
        for tj in range(nt):
            start = pl.multiple_of(_TILE * (tj + nt) - _TILE * gib, _TILE)
            out_ref[0, 0, r0:r0 + _TILE, _TILE * tj:_TILE * (tj + 1)] = (
                e_ref[:, pl.ds(start, _TILE)])


def kernel(qk_dots, table):
    del qk_dots
    tbl_t = table.T.reshape(_HEADS, 1, _NUM_BUCKETS)
    out = pl.pallas_call(
        _bias_body,
        grid=(_HEADS, _SEQ // _BLK_I),
        in_specs=[
            pl.BlockSpec((1, 1, _NUM_BUCKETS), lambda h, ib: (h, 0, 0),
                         memory_space=pltpu.SMEM),
        ],
        out_specs=pl.BlockSpec((1, 1, _BLK_I, _SEQ),
                               lambda h, ib: (0, h, ib, 0)),
        out_shape=jax.ShapeDtypeStruct((1, _HEADS, _SEQ, _SEQ), jnp.float32),
        scratch_shapes=[pltpu.VMEM((_EROWS, _W), jnp.float32)],
    )(tbl_t)
    return out


# speedup vs baseline: 1.7945x; 1.1081x over previous
import math

import jax
import jax.numpy as jnp
from jax.experimental import pallas as pl
from jax.experimental.pallas import tpu as pltpu

_SEQ = 2048
_HEADS = 16
_NUM_BUCKETS = 32
_MAX_DISTANCE = 128
_W = 2 * _SEQ
_BLK_I = 1024
_EROWS = 128
_TILE = 128


def _bias_body(tbl_ref, out_ref, e_ref):
    h = pl.program_id(0)
    ib = pl.program_id(1)

    @pl.when(ib == 0)
    def _build():
        z = jax.lax.broadcasted_iota(jnp.int32, (1, _W), 1)
        rel = z - _SEQ
        n = -rel
        half = _NUM_BUCKETS // 2
        ret = jnp.where(n < 0, half, 0)
        na = jnp.abs(n)
        max_exact = half // 2
        nsq = (na * na).astype(jnp.float32)
        e = (jax.lax.bitcast_convert_type(nsq, jnp.int32) >> 23) - 127
        val_large = jnp.minimum(max_exact + (e - 6), half - 1)
        bucket = ret + jnp.where(na < max_exact, na, val_large)
        w = jnp.zeros((1, _W), jnp.float32)
        for b in range(_NUM_BUCKETS):
            w = w + jnp.where(bucket == b, tbl_ref[0, 0, b], 0.0)
        wb = jnp.broadcast_to(w, (_EROWS, _W))
        e_ref[:, :] = pltpu.roll(wb, 0, axis=1, stride=1, stride_axis=0)

    nt = _SEQ // _TILE
    for sub in range(_BLK_I // _TILE):
        gib = (_BLK_I // _TILE) * ib + sub
        r0 = _TILE * sub
        for tj in range(nt):
            start = pl.multiple_of(_TILE * (tj + nt) - _TILE * gib, _TILE)
            out_ref[0, 0, r0:r0 + _TILE, _TILE * tj:_TILE * (tj + 1)] = (
                e_ref[:, pl.ds(start, _TILE)])


def kernel(qk_dots, table):
    del qk_dots
    tbl_t = table.T.reshape(_HEADS, 1, _NUM_BUCKETS)
    out = pl.pallas_call(
        _bias_body,
        grid=(_HEADS, _SEQ // _BLK_I),
        in_specs=[
            pl.BlockSpec((1, 1, _NUM_BUCKETS), lambda h, ib: (h, 0, 0),
                         memory_space=pltpu.SMEM),
        ],
        out_specs=pl.BlockSpec((1, 1, _BLK_I, _SEQ),
                               lambda h, ib: (0, h, ib, 0)),
        out_shape=jax.ShapeDtypeStruct((1, _HEADS, _SEQ, _SEQ), jnp.float32),
        scratch_shapes=[pltpu.VMEM((_EROWS, _W), jnp.float32)],
    )(tbl_t)
    return out


# speedup vs baseline: 1.9925x; 1.1103x over previous
import math

import jax
import jax.numpy as jnp
from jax.experimental import pallas as pl
from jax.experimental.pallas import tpu as pltpu

_SEQ = 2048
_HEADS = 16
_NUM_BUCKETS = 32
_MAX_DISTANCE = 128
_W = 2 * _SEQ
_BLK_I = 2048
_EROWS = 128
_TILE = 128


def _bias_body(tbl_ref, out_ref, e_ref):
    h = pl.program_id(0)
    ib = pl.program_id(1)

    @pl.when(ib == 0)
    def _build():
        z = jax.lax.broadcasted_iota(jnp.int32, (1, _W), 1)
        rel = z - _SEQ
        n = -rel
        half = _NUM_BUCKETS // 2
        ret = jnp.where(n < 0, half, 0)
        na = jnp.abs(n)
        max_exact = half // 2
        nsq = (na * na).astype(jnp.float32)
        e = (jax.lax.bitcast_convert_type(nsq, jnp.int32) >> 23) - 127
        val_large = jnp.minimum(max_exact + (e - 6), half - 1)
        bucket = ret + jnp.where(na < max_exact, na, val_large)
        w = jnp.zeros((1, _W), jnp.float32)
        for b in range(_NUM_BUCKETS):
            w = w + jnp.where(bucket == b, tbl_ref[0, 0, b], 0.0)
        wb = jnp.broadcast_to(w, (_EROWS, _W))
        e_ref[:, :] = pltpu.roll(wb, 0, axis=1, stride=1, stride_axis=0)

    nt = _SEQ // _TILE
    for sub in range(_BLK_I // _TILE):
        gib = (_BLK_I // _TILE) * ib + sub
        r0 = _TILE * sub
        for tj in range(nt):
            start = pl.multiple_of(_TILE * (tj + nt) - _TILE * gib, _TILE)
            out_ref[0, 0, r0:r0 + _TILE, _TILE * tj:_TILE * (tj + 1)] = (
                e_ref[:, pl.ds(start, _TILE)])


def kernel(qk_dots, table):
    del qk_dots
    tbl_t = table.T.reshape(_HEADS, 1, _NUM_BUCKETS)
    out = pl.pallas_call(
        _bias_body,
        grid=(_HEADS, _SEQ // _BLK_I),
        in_specs=[
            pl.BlockSpec((1, 1, _NUM_BUCKETS), lambda h, ib: (h, 0, 0),
                         memory_space=pltpu.SMEM),
        ],
        out_specs=pl.BlockSpec((1, 1, _BLK_I, _SEQ),
                               lambda h, ib: (0, h, ib, 0)),
        out_shape=jax.ShapeDtypeStruct((1, _HEADS, _SEQ, _SEQ), jnp.float32),
        scratch_shapes=[pltpu.VMEM((_EROWS, _W), jnp.float32)],
    )(tbl_t)
    return out
